# baseline (speedup 1.0000x reference)
import functools
import math

import jax
import jax.numpy as jnp
from jax import lax
from jax.experimental import pallas as pl
from jax.experimental.pallas import tpu as pltpu
from jax.experimental.pallas import tpu_sc as plsc

_D = 64
_SCALE = math.sqrt(_D)
_PAD = 0

_NUM_ROWS = 100000
_BATCH = 4096
_SEQ = 200
_B_TOK = _BATCH * _SEQ
_HALF = _B_TOK // 2

_info = plsc.get_sparse_core_info()
_NC = _info.num_cores
_NS = _info.num_subcores
_NW = _NC * _NS

_HPW = _HALF // _NW
_CHUNK = 256
_NCHUNK = _HPW // _CHUNK

_TROWS = _NUM_ROWS // 2
_TBLK = 2000


def _prescale_body(t_ref, o_ref):
    rows = lax.broadcasted_iota(jnp.int32, t_ref.shape, 0)
    lanes = lax.broadcasted_iota(jnp.int32, t_ref.shape, 1)
    is_pad = (rows + pl.program_id(0) * _TBLK == 0) & (lanes < _D)
    o_ref[...] = jnp.where(is_pad, 0.0, t_ref[...] * _SCALE)


def _prescale(table):
    packed = table.reshape(_TROWS, 2 * _D)
    out = pl.pallas_call(
        _prescale_body,
        grid=(_TROWS // _TBLK,),
        in_specs=[pl.BlockSpec((_TBLK, 2 * _D), lambda i: (i, 0))],
        out_specs=pl.BlockSpec((_TBLK, 2 * _D), lambda i: (i, 0)),
        out_shape=jax.ShapeDtypeStruct((_TROWS, 2 * _D), jnp.float32),
    )(packed)
    return out.reshape(_NUM_ROWS, _D)


_mesh = plsc.VectorSubcoreMesh(core_axis_name="c", subcore_axis_name="s")


@functools.partial(
    pl.kernel,
    mesh=_mesh,
    out_type=jax.ShapeDtypeStruct((_HALF, 2 * _D), jnp.float32),
    scratch_types=[
        pltpu.VMEM((_HPW,), jnp.int32),
        pltpu.VMEM((_HPW,), jnp.int32),
        pltpu.VMEM((_CHUNK, _D), jnp.float32),
        pltpu.VMEM((_CHUNK, _D), jnp.float32),
        pltpu.VMEM((_CHUNK, _D), jnp.float32),
        pltpu.VMEM((_CHUNK, _D), jnp.float32),
        pltpu.SemaphoreType.DMA,
        pltpu.SemaphoreType.DMA,
    ],
    compiler_params=pltpu.CompilerParams(use_tc_tiling_on_sc=False),
)
def _sc_gather(idxe_hbm, idxo_hbm, table_hbm, out_hbm, idxe_v, idxo_v,
               e0_v, o0_v, e1_v, o1_v, g0, g1):
    wid = lax.axis_index("s") * _NC + lax.axis_index("c")
    base = wid * _HPW
    pltpu.sync_copy(idxe_hbm.at[pl.ds(base, _HPW)], idxe_v)
    pltpu.sync_copy(idxo_hbm.at[pl.ds(base, _HPW)], idxo_v)

    def gather(ci, e_v, o_v, sem):
        off = ci * _CHUNK
        pltpu.async_copy(table_hbm.at[idxe_v.at[pl.ds(off, _CHUNK)]], e_v, sem)
        pltpu.async_copy(table_hbm.at[idxo_v.at[pl.ds(off, _CHUNK)]], o_v, sem)

    def wait_gather(ci, e_v, o_v, sem):
        off = ci * _CHUNK
        pltpu.make_async_copy(
            table_hbm.at[idxe_v.at[pl.ds(off, _CHUNK)]], e_v, sem
        ).wait()
        pltpu.make_async_copy(
            table_hbm.at[idxo_v.at[pl.ds(off, _CHUNK)]], o_v, sem
        ).wait()

    def store(ci, e_v, o_v):
        row0 = base + ci * _CHUNK
        pltpu.sync_copy(e_v, out_hbm.at[pl.ds(row0, _CHUNK), pl.ds(0, _D)])
        pltpu.sync_copy(o_v, out_hbm.at[pl.ds(row0, _CHUNK), pl.ds(_D, _D)])

    gather(0, e0_v, o0_v, g0)

    def body(p, carry):
        i = 2 * p
        gather(i + 1, e1_v, o1_v, g1)
        wait_gather(i, e0_v, o0_v, g0)
        store(i, e0_v, o0_v)

        @pl.when(i + 2 < _NCHUNK)
        def _():
            gather(i + 2, e0_v, o0_v, g0)

        wait_gather(i + 1, e1_v, o1_v, g1)
        store(i + 1, e1_v, o1_v)
        return carry

    lax.fori_loop(0, _NCHUNK // 2, body, 0)


_FB = 128
_FIN = _FB * _SEQ // 2


def _format_body(x_ref, o_ref):
    x = x_ref[...]
    e = x[:, :_D].reshape(_FB, _SEQ // 2, _D)
    o = x[:, _D:].reshape(_FB, _SEQ // 2, _D)
    et = jnp.transpose(e, (1, 2, 0))
    ot = jnp.transpose(o, (1, 2, 0))
    y = jnp.stack([et, ot], axis=1)
    o_ref[...] = y.reshape(_SEQ, _D, _FB)


def _format(wide):
    return pl.pallas_call(
        _format_body,
        grid=(_BATCH // _FB,),
        in_specs=[pl.BlockSpec((_FIN, 2 * _D), lambda i: (i, 0))],
        out_specs=pl.BlockSpec((_SEQ, _D, _FB), lambda i: (0, 0, i)),
        out_shape=jax.ShapeDtypeStruct((_SEQ, _D, _BATCH), jnp.float32),
    )(wide)


def kernel(inputs, table):
    table_scaled = _prescale(table)
    pairs = inputs.reshape(_HALF, 2).astype(jnp.int32)
    idx_e = pairs[:, 0]
    idx_o = pairs[:, 1]
    wide = _sc_gather(idx_e, idx_o, table_scaled)
    out_t = _format(wide)
    return jnp.transpose(out_t, (2, 0, 1))


# speedup vs baseline: 1.2552x; 1.2552x over previous
import functools
import math

import jax
import jax.numpy as jnp
from jax import lax
from jax.experimental import pallas as pl
from jax.experimental.pallas import tpu as pltpu
from jax.experimental.pallas import tpu_sc as plsc

_D = 64
_SCALE = math.sqrt(_D)
_PAD = 0

_NUM_ROWS = 100000
_BATCH = 4096
_SEQ = 200
_B_TOK = _BATCH * _SEQ
_HALF = _B_TOK // 2

_info = plsc.get_sparse_core_info()
_NC = _info.num_cores
_NS = _info.num_subcores
_NW = _NC * _NS

_HPW = _HALF // _NW
_CHUNK = 256
_NCHUNK = _HPW // _CHUNK

_TROWS = _NUM_ROWS // 2
_TBLK = 2000


def _prescale_body(t_ref, o_ref):
    rows = lax.broadcasted_iota(jnp.int32, t_ref.shape, 0)
    lanes = lax.broadcasted_iota(jnp.int32, t_ref.shape, 1)
    is_pad = (rows + pl.program_id(0) * _TBLK == 0) & (lanes < _D)
    o_ref[...] = jnp.where(is_pad, 0.0, t_ref[...] * _SCALE)


def _prescale(table):
    packed = table.reshape(_TROWS, 2 * _D)
    out = pl.pallas_call(
        _prescale_body,
        grid=(_TROWS // _TBLK,),
        in_specs=[pl.BlockSpec((_TBLK, 2 * _D), lambda i: (i, 0))],
        out_specs=pl.BlockSpec((_TBLK, 2 * _D), lambda i: (i, 0)),
        out_shape=jax.ShapeDtypeStruct((_TROWS, 2 * _D), jnp.float32),
    )(packed)
    return out.reshape(_NUM_ROWS, _D)


_mesh = plsc.VectorSubcoreMesh(core_axis_name="c", subcore_axis_name="s")


@functools.partial(
    pl.kernel,
    mesh=_mesh,
    out_type=jax.ShapeDtypeStruct((_HALF, 2 * _D), jnp.float32),
    scratch_types=[
        pltpu.VMEM((2 * _HPW,), jnp.int32),
        pltpu.VMEM((_CHUNK, _D), jnp.float32),
        pltpu.VMEM((_CHUNK, _D), jnp.float32),
        pltpu.VMEM((_CHUNK, _D), jnp.float32),
        pltpu.VMEM((_CHUNK, _D), jnp.float32),
        pltpu.SemaphoreType.DMA,
        pltpu.SemaphoreType.DMA,
    ],
    compiler_params=pltpu.CompilerParams(use_tc_tiling_on_sc=False),
)
def _sc_gather(idx_hbm, table_hbm, out_hbm, idx_v, e0_v, o0_v, e1_v, o1_v,
               g0, g1):
    wid = lax.axis_index("s") * _NC + lax.axis_index("c")
    base = wid * _HPW
    pltpu.sync_copy(idx_hbm.at[pl.ds(2 * base, 2 * _HPW)], idx_v)

    def gather(ci, e_v, o_v, sem):
        off = ci * _CHUNK
        pltpu.async_copy(table_hbm.at[idx_v.at[pl.ds(off, _CHUNK)]], e_v, sem)
        pltpu.async_copy(
            table_hbm.at[idx_v.at[pl.ds(_HPW + off, _CHUNK)]], o_v, sem
        )

    def wait_gather(ci, e_v, o_v, sem):
        off = ci * _CHUNK
        pltpu.make_async_copy(
            table_hbm.at[idx_v.at[pl.ds(off, _CHUNK)]], e_v, sem
        ).wait()
        pltpu.make_async_copy(
            table_hbm.at[idx_v.at[pl.ds(_HPW + off, _CHUNK)]], o_v, sem
        ).wait()

    def store(ci, e_v, o_v):
        row0 = base + ci * _CHUNK
        pltpu.sync_copy(e_v, out_hbm.at[pl.ds(row0, _CHUNK), pl.ds(0, _D)])
        pltpu.sync_copy(o_v, out_hbm.at[pl.ds(row0, _CHUNK), pl.ds(_D, _D)])

    gather(0, e0_v, o0_v, g0)

    def body(p, carry):
        i = 2 * p
        gather(i + 1, e1_v, o1_v, g1)
        wait_gather(i, e0_v, o0_v, g0)
        store(i, e0_v, o0_v)

        @pl.when(i + 2 < _NCHUNK)
        def _():
            gather(i + 2, e0_v, o0_v, g0)

        wait_gather(i + 1, e1_v, o1_v, g1)
        store(i + 1, e1_v, o1_v)
        return carry

    lax.fori_loop(0, _NCHUNK // 2, body, 0)


_FB = 128
_FH = _FB // 2
_FIN = _HPW


def _format_body(x_ref, o_ref):
    x = x_ref[...]
    e = x[:, :_D].reshape(_FH, _SEQ, _D)
    o = x[:, _D:].reshape(_FH, _SEQ, _D)
    et = jnp.transpose(e, (1, 2, 0))
    ot = jnp.transpose(o, (1, 2, 0))
    o_ref[...] = jnp.concatenate([et, ot], axis=2)


def _format(wide):
    return pl.pallas_call(
        _format_body,
        grid=(_BATCH // _FB,),
        in_specs=[pl.BlockSpec((_FIN, 2 * _D), lambda i: (i, 0))],
        out_specs=pl.BlockSpec((_SEQ, _D, _FB), lambda i: (0, 0, i)),
        out_shape=jax.ShapeDtypeStruct((_SEQ, _D, _BATCH), jnp.float32),
    )(wide)


def kernel(inputs, table):
    table_scaled = _prescale(table)
    idx = inputs.reshape(-1).astype(jnp.int32)
    wide = _sc_gather(idx, table_scaled)
    out_t = _format(wide)
    return jnp.transpose(out_t, (2, 0, 1))


# speedup vs baseline: 1.6135x; 1.2855x over previous
import functools
import math

import jax
import jax.numpy as jnp
from jax import lax
from jax.experimental import pallas as pl
from jax.experimental.pallas import tpu as pltpu
from jax.experimental.pallas import tpu_sc as plsc

_D = 64
_SCALE = math.sqrt(_D)
_PAD = 0

_NUM_ROWS = 100000
_BATCH = 4096
_SEQ = 200
_B_TOK = _BATCH * _SEQ

_info = plsc.get_sparse_core_info()
_NC = _info.num_cores
_NS = _info.num_subcores
_NW = _NC * _NS

_BPW = _B_TOK // _NW
_CHUNK = 512
_NCHUNK = _BPW // _CHUNK

_TROWS = _NUM_ROWS // 2
_TBLK = 2000


def _prescale_body(t_ref, o_ref):
    rows = lax.broadcasted_iota(jnp.int32, t_ref.shape, 0)
    lanes = lax.broadcasted_iota(jnp.int32, t_ref.shape, 1)
    is_pad = (rows + pl.program_id(0) * _TBLK == 0) & (lanes < _D)
    o_ref[...] = jnp.where(is_pad, 0.0, t_ref[...] * _SCALE)


def _prescale(table):
    packed = table.reshape(_TROWS, 2 * _D)
    out = pl.pallas_call(
        _prescale_body,
        grid=(_TROWS // _TBLK,),
        in_specs=[pl.BlockSpec((_TBLK, 2 * _D), lambda i: (i, 0))],
        out_specs=pl.BlockSpec((_TBLK, 2 * _D), lambda i: (i, 0)),
        out_shape=jax.ShapeDtypeStruct((_TROWS, 2 * _D), jnp.float32),
    )(packed)
    return out.reshape(_NUM_ROWS, _D)


_mesh = plsc.VectorSubcoreMesh(core_axis_name="c", subcore_axis_name="s")


@functools.partial(
    pl.kernel,
    mesh=_mesh,
    out_type=jax.ShapeDtypeStruct((_B_TOK, 2 * _D), jnp.float32),
    scratch_types=[
        pltpu.VMEM((_BPW,), jnp.int32),
        pltpu.VMEM((_CHUNK, _D), jnp.float32),
        pltpu.VMEM((_CHUNK, _D), jnp.float32),
        pltpu.SemaphoreType.DMA,
        pltpu.SemaphoreType.DMA,
    ],
    compiler_params=pltpu.CompilerParams(use_tc_tiling_on_sc=False),
)
def _sc_gather(idx_hbm, table_hbm, out_hbm, idx_v, rows0_v, rows1_v, g0, g1):
    wid = lax.axis_index("s") * _NC + lax.axis_index("c")
    base = wid * _BPW
    pltpu.sync_copy(idx_hbm.at[pl.ds(base, _BPW)], idx_v)

    def gather(ci, rows_v, sem):
        off = ci * _CHUNK
        pltpu.async_copy(table_hbm.at[idx_v.at[pl.ds(off, _CHUNK)]], rows_v, sem)

    def wait_gather(ci, rows_v, sem):
        off = ci * _CHUNK
        pltpu.make_async_copy(
            table_hbm.at[idx_v.at[pl.ds(off, _CHUNK)]], rows_v, sem
        ).wait()

    def store(ci, rows_v):
        pltpu.sync_copy(
            rows_v,
            out_hbm.at[pl.ds(base + ci * _CHUNK, _CHUNK), pl.ds(0, _D)],
        )

    gather(0, rows0_v, g0)

    def body(p, carry):
        i = 2 * p
        gather(i + 1, rows1_v, g1)
        wait_gather(i, rows0_v, g0)
        store(i, rows0_v)

        @pl.when(i + 2 < _NCHUNK)
        def _():
            gather(i + 2, rows0_v, g0)

        wait_gather(i + 1, rows1_v, g1)
        store(i + 1, rows1_v)
        return carry

    lax.fori_loop(0, _NCHUNK // 2, body, 0)


_FB = 128
_FIN = _FB * _SEQ


def _format_body(x_ref, o_ref):
    x = x_ref[...]
    v = x[:, :_D].reshape(_FB, _SEQ, _D)
    o_ref[...] = jnp.transpose(v, (1, 2, 0))


def _format(wide):
    return pl.pallas_call(
        _format_body,
        grid=(_BATCH // _FB,),
        in_specs=[pl.BlockSpec((_FIN, 2 * _D), lambda i: (i, 0))],
        out_specs=pl.BlockSpec((_SEQ, _D, _FB), lambda i: (0, 0, i)),
        out_shape=jax.ShapeDtypeStruct((_SEQ, _D, _BATCH), jnp.float32),
    )(wide)


def kernel(inputs, table):
    table_scaled = _prescale(table)
    idx = inputs.reshape(-1).astype(jnp.int32)
    wide = _sc_gather(idx, table_scaled)
    out_t = _format(wide)
    return jnp.transpose(out_t, (2, 0, 1))
